# Initial kernel scaffold; baseline (speedup 1.0000x reference)
import functools

import jax
import jax.numpy as jnp
from jax import lax
from jax.experimental import pallas as pl
from jax.experimental.pallas import tpu as pltpu
from jax.experimental.pallas import tpu_sc as plsc

V = 1_000_000
D = 64
RANK = 16
SCALE = 2.0
B_TOK = 16384 * 50

VBLK = 32768
NVBLK = pl.cdiv(V, VBLK)


def _weff_body(w_ref, a_ref, bt_ref, o_ref):
    a = a_ref[...]
    bt = bt_ref[...]
    delta = lax.dot_general(a, bt, (((0,), (0,)), ((), ())),
                            preferred_element_type=jnp.float32)
    o_ref[...] = w_ref[...] + delta


def _weff(weight, lora_a, bt_scaled):
    return pl.pallas_call(
        _weff_body,
        grid=(NVBLK,),
        in_specs=[
            pl.BlockSpec((VBLK, D), lambda i: (i, 0)),
            pl.BlockSpec((RANK, VBLK), lambda i: (0, i)),
            pl.BlockSpec((RANK, D), lambda i: (0, 0)),
        ],
        out_specs=pl.BlockSpec((VBLK, D), lambda i: (i, 0)),
        out_shape=jax.ShapeDtypeStruct((V, D), jnp.float32),
    )(weight, lora_a, bt_scaled)


_info = plsc.get_sparse_core_info()
NC, NS = _info.num_cores, _info.num_subcores
NW = NC * NS
IDXW = 128
ROWS_TOT = B_TOK // IDXW
ROWS_PER_W = ROWS_TOT // NW
K = 4
CH = K * IDXW
NCH = ROWS_PER_W // K


def _sc_gather_body(weff_hbm, idx_hbm, out_hbm, idx_v, rows_v, gsem):
    wid = lax.axis_index("s") * NC + lax.axis_index("c")
    row0 = wid * ROWS_PER_W
    pltpu.sync_copy(idx_hbm.at[pl.ds(row0, ROWS_PER_W)], idx_v)

    def chunk(c, carry):
        r = c * K
        cps = [
            pltpu.async_copy(weff_hbm.at[idx_v.at[r + j]],
                             rows_v.at[pl.ds(j * IDXW, IDXW)], gsem)
            for j in range(K)
        ]
        for cp in cps:
            cp.wait()
        pltpu.sync_copy(rows_v, out_hbm.at[pl.ds((row0 + r) * IDXW, CH)])
        return carry

    lax.fori_loop(0, NCH, chunk, 0)


_sc_gather = pl.kernel(
    _sc_gather_body,
    mesh=plsc.VectorSubcoreMesh(core_axis_name="c", subcore_axis_name="s"),
    out_type=jax.ShapeDtypeStruct((B_TOK, D), jnp.float32),
    scratch_types=[
        pltpu.VMEM((ROWS_PER_W, IDXW), jnp.int32),
        pltpu.VMEM((CH, D), jnp.float32),
        pltpu.SemaphoreType.DMA,
    ],
)


def kernel(x, weight, lora_a, lora_b):
    xf = x.reshape(-1).astype(jnp.int32).reshape(ROWS_TOT, IDXW)
    bt_scaled = lora_b.T * SCALE
    weff = _weff(weight, lora_a, bt_scaled)
    out = _sc_gather(weff, xf)
    return out.reshape(x.shape + (D,))


# speedup vs baseline: 11.3695x; 11.3695x over previous
import functools

import jax
import jax.numpy as jnp
from jax import lax
from jax.experimental import pallas as pl
from jax.experimental.pallas import tpu as pltpu
from jax.experimental.pallas import tpu_sc as plsc

V = 1_000_000
D = 64
RANK = 16
SCALE = 2.0
B_TOK = 16384 * 50

VBLK = 16384
NVBLK = pl.cdiv(V, VBLK)


def _weff_body(w_ref, a_ref, bt_ref, o_ref):
    a = a_ref[...]
    bt = bt_ref[...]
    delta = lax.dot_general(a, bt, (((0,), (0,)), ((), ())),
                            preferred_element_type=jnp.float32)
    s = w_ref[...] + delta
    o_ref[...] = jnp.concatenate([s, jnp.zeros_like(s)], axis=1)


def _weff(weight, lora_a, bt_scaled):
    return pl.pallas_call(
        _weff_body,
        grid=(NVBLK,),
        in_specs=[
            pl.BlockSpec((VBLK, D), lambda i: (i, 0)),
            pl.BlockSpec((RANK, VBLK), lambda i: (0, i)),
            pl.BlockSpec((RANK, D), lambda i: (0, 0)),
        ],
        out_specs=pl.BlockSpec((VBLK, 2 * D), lambda i: (i, 0)),
        out_shape=jax.ShapeDtypeStruct((V, 2 * D), jnp.float32),
    )(weight, lora_a, bt_scaled)


_info = plsc.get_sparse_core_info()
NC, NS = _info.num_cores, _info.num_subcores
NW = NC * NS
IDXW = 128
ROWS_TOT = B_TOK // IDXW
ROWS_PER_W = ROWS_TOT // NW
K = 4
CH = K * IDXW
NCH = ROWS_PER_W // K


def _sc_gather_body(weff_hbm, idx_hbm, out_hbm, idx_v, rows_v, gsem):
    wid = lax.axis_index("s") * NC + lax.axis_index("c")
    row0 = wid * ROWS_PER_W
    pltpu.sync_copy(idx_hbm.at[pl.ds(row0, ROWS_PER_W)], idx_v)

    def chunk(c, carry):
        r = c * K
        cps = [
            pltpu.async_copy(weff_hbm.at[idx_v.at[r + j]],
                             rows_v.at[pl.ds(j * IDXW, IDXW)], gsem)
            for j in range(K)
        ]
        for cp in cps:
            cp.wait()
        pltpu.sync_copy(rows_v, out_hbm.at[pl.ds((row0 + r) * IDXW, CH)])
        return carry

    lax.fori_loop(0, NCH, chunk, 0)


_sc_gather = pl.kernel(
    _sc_gather_body,
    mesh=plsc.VectorSubcoreMesh(core_axis_name="c", subcore_axis_name="s"),
    out_type=jax.ShapeDtypeStruct((B_TOK, 2 * D), jnp.float32),
    scratch_types=[
        pltpu.VMEM((ROWS_PER_W, IDXW), jnp.int32),
        pltpu.VMEM((CH, 2 * D), jnp.float32),
        pltpu.SemaphoreType.DMA,
    ],
)


def kernel(x, weight, lora_a, lora_b):
    xf = x.reshape(-1).astype(jnp.int32).reshape(ROWS_TOT, IDXW)
    bt_scaled = lora_b.T * SCALE
    weff = _weff(weight, lora_a, bt_scaled)
    out = _sc_gather(weff, xf)
    return out[:, :D].reshape(x.shape + (D,))


# speedup vs baseline: 15.2209x; 1.3387x over previous
import functools

import jax
import jax.numpy as jnp
from jax import lax
from jax.experimental import pallas as pl
from jax.experimental.pallas import tpu as pltpu
from jax.experimental.pallas import tpu_sc as plsc

V = 1_000_000
D = 64
RANK = 16
SCALE = 2.0
B_TOK = 16384 * 50

VBLK = 16384
NVBLK = pl.cdiv(V, VBLK)


def _weff_body(wt_ref, a_ref, p_ref, o_ref):
    m = jnp.concatenate([a_ref[...], wt_ref[...]], axis=0)
    o_ref[...] = lax.dot_general(m, p_ref[...], (((0,), (0,)), ((), ())),
                                 preferred_element_type=jnp.float32)


def _weff(weight_t, lora_a, proj):
    return pl.pallas_call(
        _weff_body,
        grid=(NVBLK,),
        in_specs=[
            pl.BlockSpec((D, VBLK), lambda i: (0, i)),
            pl.BlockSpec((RANK, VBLK), lambda i: (0, i)),
            pl.BlockSpec((RANK + D, 2 * D), lambda i: (0, 0)),
        ],
        out_specs=pl.BlockSpec((VBLK, 2 * D), lambda i: (i, 0)),
        out_shape=jax.ShapeDtypeStruct((V, 2 * D), jnp.float32),
    )(weight_t, lora_a, proj)


_info = plsc.get_sparse_core_info()
NC, NS = _info.num_cores, _info.num_subcores
NW = NC * NS
IDXW = 128
ROWS_TOT = B_TOK // IDXW
ROWS_PER_W = ROWS_TOT // NW
K = 4
CH = K * IDXW
NCH = ROWS_PER_W // K


def _sc_gather_body(weff_hbm, idx_hbm, out_hbm, idx_v, rows_v, gsem):
    wid = lax.axis_index("s") * NC + lax.axis_index("c")
    row0 = wid * ROWS_PER_W
    pltpu.sync_copy(idx_hbm.at[pl.ds(row0, ROWS_PER_W)], idx_v)

    def chunk(c, carry):
        r = c * K
        cps = [
            pltpu.async_copy(weff_hbm.at[idx_v.at[r + j]],
                             rows_v.at[pl.ds(j * IDXW, IDXW)], gsem)
            for j in range(K)
        ]
        for cp in cps:
            cp.wait()
        pltpu.sync_copy(rows_v, out_hbm.at[pl.ds((row0 + r) * IDXW, CH)])
        return carry

    lax.fori_loop(0, NCH, chunk, 0)


_sc_gather = pl.kernel(
    _sc_gather_body,
    mesh=plsc.VectorSubcoreMesh(core_axis_name="c", subcore_axis_name="s"),
    out_type=jax.ShapeDtypeStruct((B_TOK, 2 * D), jnp.float32),
    scratch_types=[
        pltpu.VMEM((ROWS_PER_W, IDXW), jnp.int32),
        pltpu.VMEM((CH, 2 * D), jnp.float32),
        pltpu.SemaphoreType.DMA,
    ],
)


def kernel(x, weight, lora_a, lora_b):
    xf = x.reshape(-1).astype(jnp.int32).reshape(ROWS_TOT, IDXW)
    proj = jnp.zeros((RANK + D, 2 * D), jnp.float32)
    proj = proj.at[:RANK, :D].set(lora_b.T * SCALE)
    proj = proj.at[RANK:, :D].set(jnp.eye(D, dtype=jnp.float32))
    weff = _weff(weight.T, lora_a, proj)
    out = _sc_gather(weff, xf)
    return out[:, :D].reshape(x.shape + (D,))


# speedup vs baseline: 15.2913x; 1.0046x over previous
import functools

import jax
import jax.numpy as jnp
from jax import lax
from jax.experimental import pallas as pl
from jax.experimental.pallas import tpu as pltpu
from jax.experimental.pallas import tpu_sc as plsc

V = 1_000_000
D = 64
RANK = 16
SCALE = 2.0
B_TOK = 16384 * 50

VBLK = 32768
NVBLK = pl.cdiv(V, VBLK)


def _weff_body(wt_ref, a_ref, p_ref, o_ref):
    m = jnp.concatenate([a_ref[...], wt_ref[...]], axis=0)
    o_ref[...] = lax.dot_general(m, p_ref[...], (((0,), (0,)), ((), ())),
                                 preferred_element_type=jnp.float32)


def _weff(weight_t, lora_a, proj):
    return pl.pallas_call(
        _weff_body,
        grid=(NVBLK,),
        in_specs=[
            pl.BlockSpec((D, VBLK), lambda i: (0, i)),
            pl.BlockSpec((RANK, VBLK), lambda i: (0, i)),
            pl.BlockSpec((RANK + D, 2 * D), lambda i: (0, 0)),
        ],
        out_specs=pl.BlockSpec((VBLK, 2 * D), lambda i: (i, 0)),
        out_shape=jax.ShapeDtypeStruct((V, 2 * D), jnp.float32),
        compiler_params=pltpu.CompilerParams(
            dimension_semantics=("arbitrary",)),
    )(weight_t, lora_a, proj)


_info = plsc.get_sparse_core_info()
NC, NS = _info.num_cores, _info.num_subcores
NW = NC * NS
IDXW = 128
ROWS_TOT = B_TOK // IDXW
ROWS_PER_W = ROWS_TOT // NW
K = 4
CH = K * IDXW
NCH = ROWS_PER_W // K


def _sc_gather_body(weff_hbm, idx_hbm, out_hbm, idx_v, rows_v, gsem):
    wid = lax.axis_index("s") * NC + lax.axis_index("c")
    row0 = wid * ROWS_PER_W
    pltpu.sync_copy(idx_hbm.at[pl.ds(row0, ROWS_PER_W)], idx_v)

    def chunk(c, carry):
        r = c * K
        cps = [
            pltpu.async_copy(weff_hbm.at[idx_v.at[r + j]],
                             rows_v.at[pl.ds(j * IDXW, IDXW)], gsem)
            for j in range(K)
        ]
        for cp in cps:
            cp.wait()
        pltpu.sync_copy(rows_v, out_hbm.at[pl.ds((row0 + r) * IDXW, CH)])
        return carry

    lax.fori_loop(0, NCH, chunk, 0)


_sc_gather = pl.kernel(
    _sc_gather_body,
    mesh=plsc.VectorSubcoreMesh(core_axis_name="c", subcore_axis_name="s"),
    out_type=jax.ShapeDtypeStruct((B_TOK, 2 * D), jnp.float32),
    scratch_types=[
        pltpu.VMEM((ROWS_PER_W, IDXW), jnp.int32),
        pltpu.VMEM((CH, 2 * D), jnp.float32),
        pltpu.SemaphoreType.DMA,
    ],
)


def kernel(x, weight, lora_a, lora_b):
    xf = x.reshape(-1).astype(jnp.int32).reshape(ROWS_TOT, IDXW)
    proj = jnp.zeros((RANK + D, 2 * D), jnp.float32)
    proj = proj.at[:RANK, :D].set(lora_b.T * SCALE)
    proj = proj.at[RANK:, :D].set(jnp.eye(D, dtype=jnp.float32))
    weff = _weff(weight.T, lora_a, proj)
    out = _sc_gather(weff, xf)
    return out[:, :D].reshape(x.shape + (D,))


# speedup vs baseline: 15.5843x; 1.0192x over previous
import functools

import jax
import jax.numpy as jnp
from jax import lax
from jax.experimental import pallas as pl
from jax.experimental.pallas import tpu as pltpu
from jax.experimental.pallas import tpu_sc as plsc

V = 1_000_000
D = 64
RANK = 16
SCALE = 2.0
B_TOK = 16384 * 50

VBLK = 32768
NVBLK = pl.cdiv(V, VBLK)


def _weff_body(wt_ref, a_ref, p_ref, o_ref):
    m = jnp.concatenate([a_ref[...], wt_ref[...]], axis=0)
    o_ref[...] = lax.dot_general(m, p_ref[...], (((0,), (0,)), ((), ())),
                                 preferred_element_type=jnp.float32)


def _weff(weight_t, lora_a, proj):
    return pl.pallas_call(
        _weff_body,
        grid=(NVBLK,),
        in_specs=[
            pl.BlockSpec((D, VBLK), lambda i: (0, i)),
            pl.BlockSpec((RANK, VBLK), lambda i: (0, i)),
            pl.BlockSpec((RANK + D, 2 * D), lambda i: (0, 0)),
        ],
        out_specs=pl.BlockSpec((VBLK, 2 * D), lambda i: (i, 0)),
        out_shape=jax.ShapeDtypeStruct((V, 2 * D), jnp.float32),
        compiler_params=pltpu.CompilerParams(
            dimension_semantics=("arbitrary",)),
    )(weight_t, lora_a, proj)


_info = plsc.get_sparse_core_info()
NC, NS = _info.num_cores, _info.num_subcores
NW = NC * NS
IDXW = 128
ROWS_TOT = B_TOK // IDXW
ROWS_PER_W = ROWS_TOT // NW
K = 2
CH = K * IDXW
NCH = ROWS_PER_W // K


def _sc_gather_body(weff_hbm, idx_hbm, out_hbm, idx_v, rows_v, gsem,
                    osem0, osem1):
    wid = lax.axis_index("s") * NC + lax.axis_index("c")
    row0 = wid * ROWS_PER_W
    pltpu.sync_copy(idx_hbm.at[pl.ds(row0, ROWS_PER_W)], idx_v)
    osems = (osem0, osem1)

    def start_gathers(c, b):
        r = c * K
        for j in range(K):
            pltpu.async_copy(weff_hbm.at[idx_v.at[r + j]],
                             rows_v.at[b, pl.ds(j * IDXW, IDXW)], gsem)

    def wait_gathers(c, b):
        r = c * K
        for j in range(K):
            pltpu.make_async_copy(weff_hbm.at[idx_v.at[r + j]],
                                  rows_v.at[b, pl.ds(j * IDXW, IDXW)],
                                  gsem).wait()

    def out_copy(c, b):
        return pltpu.make_async_copy(
            rows_v.at[b], out_hbm.at[pl.ds((row0 + c * K) * IDXW, CH)],
            osems[b])

    def start_out(c, b):
        pltpu.async_copy(rows_v.at[b],
                         out_hbm.at[pl.ds((row0 + c * K) * IDXW, CH)],
                         osems[b])

    def half(c, b):
        @pl.when(c >= 2)
        def _():
            out_copy(c - 2, b).wait()
        start_gathers(c, b)

        @pl.when(c >= 1)
        def _():
            wait_gathers(c - 1, 1 - b)
            start_out(c - 1, 1 - b)

    def pair(i, carry):
        half(2 * i, 0)
        half(2 * i + 1, 1)
        return carry

    lax.fori_loop(0, NCH // 2, pair, 0)
    wait_gathers(NCH - 1, 1)
    start_out(NCH - 1, 1)
    out_copy(NCH - 2, 0).wait()
    out_copy(NCH - 1, 1).wait()


_sc_gather = pl.kernel(
    _sc_gather_body,
    mesh=plsc.VectorSubcoreMesh(core_axis_name="c", subcore_axis_name="s"),
    out_type=jax.ShapeDtypeStruct((B_TOK, 2 * D), jnp.float32),
    scratch_types=[
        pltpu.VMEM((ROWS_PER_W, IDXW), jnp.int32),
        pltpu.VMEM((2, CH, 2 * D), jnp.float32),
        pltpu.SemaphoreType.DMA,
        pltpu.SemaphoreType.DMA,
        pltpu.SemaphoreType.DMA,
    ],
)


def kernel(x, weight, lora_a, lora_b):
    xf = x.reshape(-1).astype(jnp.int32).reshape(ROWS_TOT, IDXW)
    proj = jnp.zeros((RANK + D, 2 * D), jnp.float32)
    proj = proj.at[:RANK, :D].set(lora_b.T * SCALE)
    proj = proj.at[RANK:, :D].set(jnp.eye(D, dtype=jnp.float32))
    weff = _weff(weight.T, lora_a, proj)
    out = _sc_gather(weff, xf)
    return out[:, :D].reshape(x.shape + (D,))


# speedup vs baseline: 21.0191x; 1.3487x over previous
import functools

import jax
import jax.numpy as jnp
from jax import lax
from jax.experimental import pallas as pl
from jax.experimental.pallas import tpu as pltpu
from jax.experimental.pallas import tpu_sc as plsc

V = 1_000_000
D = 64
RANK = 16
SCALE = 2.0
B_TOK = 16384 * 50

VBLK = 32768
VBLK2 = VBLK // 2
NVBLK = pl.cdiv(V, VBLK)


def _weff_body(wt1_ref, wt2_ref, a1_ref, a2_ref, pl_ref, pr_ref, o_ref):
    m1 = jnp.concatenate([a1_ref[...], wt1_ref[...]], axis=0)
    m2 = jnp.concatenate([a2_ref[...], wt2_ref[...]], axis=0)
    base2 = (2 * pl.program_id(0) + 1) * VBLK2
    lane = lax.broadcasted_iota(jnp.int32, (RANK + D, VBLK2), 1)
    m2 = jnp.where(base2 + lane < V, m2, 0.0)
    h1 = lax.dot_general(m1, pl_ref[...], (((0,), (0,)), ((), ())),
                         preferred_element_type=jnp.float32)
    h2 = lax.dot_general(m2, pr_ref[...], (((0,), (0,)), ((), ())),
                         preferred_element_type=jnp.float32)
    o_ref[...] = h1 + h2


def _weff(weight_t, lora_a, proj_l, proj_r):
    return pl.pallas_call(
        _weff_body,
        grid=(NVBLK,),
        in_specs=[
            pl.BlockSpec((D, VBLK2), lambda i: (0, 2 * i)),
            pl.BlockSpec((D, VBLK2), lambda i: (0, 2 * i + 1)),
            pl.BlockSpec((RANK, VBLK2), lambda i: (0, 2 * i)),
            pl.BlockSpec((RANK, VBLK2), lambda i: (0, 2 * i + 1)),
            pl.BlockSpec((RANK + D, 2 * D), lambda i: (0, 0)),
            pl.BlockSpec((RANK + D, 2 * D), lambda i: (0, 0)),
        ],
        out_specs=pl.BlockSpec((VBLK2, 2 * D), lambda i: (i, 0)),
        out_shape=jax.ShapeDtypeStruct((NVBLK * VBLK2, 2 * D), jnp.float32),
        compiler_params=pltpu.CompilerParams(
            dimension_semantics=("arbitrary",)),
    )(weight_t, weight_t, lora_a, lora_a, proj_l, proj_r)


_info = plsc.get_sparse_core_info()
NC, NS = _info.num_cores, _info.num_subcores
NW = NC * NS
IDXW = 128
ROWS_TOT = B_TOK // IDXW
ROWS_PER_W = ROWS_TOT // NW
K = 2
CH = K * IDXW
NCH = ROWS_PER_W // K


ROW_W = D


def _sc_gather_body(weff_hbm, idx_hbm, out_hbm, idx_v, rows_v, gsem,
                    osem0, osem1):
    wid = lax.axis_index("s") * NC + lax.axis_index("c")
    row0 = wid * ROWS_PER_W
    pltpu.sync_copy(idx_hbm.at[pl.ds(row0, ROWS_PER_W)], idx_v)
    osems = (osem0, osem1)

    def start_gathers(c, b):
        r = c * K
        for j in range(K):
            pltpu.async_copy(weff_hbm.at[idx_v.at[r + j]],
                             rows_v.at[b, pl.ds(j * IDXW, IDXW)], gsem)

    def wait_gathers(c, b):
        r = c * K
        for j in range(K):
            pltpu.make_async_copy(weff_hbm.at[idx_v.at[r + j]],
                                  rows_v.at[b, pl.ds(j * IDXW, IDXW)],
                                  gsem).wait()

    def out_copy(c, b):
        return pltpu.make_async_copy(
            rows_v.at[b], out_hbm.at[pl.ds((row0 + c * K) * IDXW, CH)],
            osems[b])

    def start_out(c, b):
        pltpu.async_copy(rows_v.at[b],
                         out_hbm.at[pl.ds((row0 + c * K) * IDXW, CH)],
                         osems[b])

    def half(c, b):
        @pl.when(c >= 2)
        def _():
            out_copy(c - 2, b).wait()
        start_gathers(c, b)

        @pl.when(c >= 1)
        def _():
            wait_gathers(c - 1, 1 - b)
            start_out(c - 1, 1 - b)

    def pair(i, carry):
        half(2 * i, 0)
        half(2 * i + 1, 1)
        return carry

    lax.fori_loop(0, NCH // 2, pair, 0)
    wait_gathers(NCH - 1, 1)
    start_out(NCH - 1, 1)
    out_copy(NCH - 2, 0).wait()
    out_copy(NCH - 1, 1).wait()


_sc_gather = pl.kernel(
    _sc_gather_body,
    mesh=plsc.VectorSubcoreMesh(core_axis_name="c", subcore_axis_name="s"),
    out_type=jax.ShapeDtypeStruct((B_TOK, ROW_W), jnp.float32),
    scratch_types=[
        pltpu.VMEM((ROWS_PER_W, IDXW), jnp.int32),
        pltpu.VMEM((2, CH, ROW_W), jnp.float32),
        pltpu.SemaphoreType.DMA,
        pltpu.SemaphoreType.DMA,
        pltpu.SemaphoreType.DMA,
    ],
    compiler_params=pltpu.CompilerParams(use_tc_tiling_on_sc=False),
)


def kernel(x, weight, lora_a, lora_b):
    v = x.reshape(-1).astype(jnp.int32)
    r = ((v >> 15) << 15) + 2 * (v & (VBLK2 - 1)) + ((v >> 14) & 1)
    xf = r.reshape(ROWS_TOT, IDXW)
    pz = jnp.concatenate([lora_b.T * SCALE,
                          jnp.eye(D, dtype=jnp.float32)], axis=0)
    z = jnp.zeros_like(pz)
    weff = _weff(weight.T, lora_a,
                 jnp.concatenate([pz, z], axis=1),
                 jnp.concatenate([z, pz], axis=1)).reshape(NVBLK * VBLK, D)
    out = _sc_gather(weff, xf)
    return out.reshape(x.shape + (D,))
